# baseline (speedup 1.0000x reference)
import functools

import jax
import jax.numpy as jnp
from jax import lax
from jax.experimental import pallas as pl
from jax.experimental.pallas import tpu as pltpu
from jax.experimental.pallas import tpu_sc as plsc

N = 10000
E = 320000
D = 128
MSG = 128

NC = 2
NS = 16
NW = NC * NS

IR = 128
KC = 2
CH = KC * IR
NH = 2
E2 = E // NH
NCHH = E2 // CH

NLOC = N // NC
NACC = NLOC + 8
DUMMY = NLOC
RPT = 312
TAIL_OFF = (NS - 1) * RPT
TAIL_W = NLOC - TAIL_OFF
ZB = 104
TAIL_ZREM = NACC - TAIL_OFF - 3 * ZB

_mesh = plsc.VectorSubcoreMesh(core_axis_name="c", subcore_axis_name="s")


@functools.partial(
    pl.kernel,
    out_type=jax.ShapeDtypeStruct((E2, D), jnp.float32),
    mesh=_mesh,
    scratch_types=[
        pltpu.VMEM((KC, IR), jnp.int32),
        pltpu.VMEM((KC, IR), jnp.int32),
        pltpu.VMEM((CH, D), jnp.float32),
        pltpu.VMEM((CH, D), jnp.float32),
        pltpu.SemaphoreType.DMA,
        pltpu.SemaphoreType.DMA,
        pltpu.SemaphoreType.DMA,
        pltpu.SemaphoreType.DMA,
        pltpu.SemaphoreType.DMA,
        pltpu.SemaphoreType.DMA,
    ],
)
def _gather(
    x_hbm, src_hbm, out_hbm,
    idx_a, idx_b, rows_a, rows_b,
    semi_a, semi_b, semg_a, semg_b, semo_a, semo_b,
):
    c = lax.axis_index("c")
    s = lax.axis_index("s")
    wid = s * NC + c
    n = (NCHH - 1 - wid) // NW + 1

    bufs = (
        (idx_a, rows_a, semi_a, semg_a, semo_a),
        (idx_b, rows_b, semi_b, semg_b, semo_b),
    )

    def ch_of(k):
        return wid + k * NW

    def issue_idx(k, buf):
        pltpu.async_copy(src_hbm.at[ch_of(k)], buf[0], buf[2])

    def fire_gathers(buf):
        for j in range(KC):
            pltpu.async_copy(
                x_hbm.at[buf[0].at[j]], buf[1].at[pl.ds(j * IR, IR)], buf[3]
            )

    def wait_idx(buf):
        pltpu.make_async_copy(src_hbm.at[0], buf[0], buf[2]).wait()

    def wait_gathers(buf):
        for j in range(KC):
            pltpu.make_async_copy(
                x_hbm.at[pl.ds(0, IR)], buf[1].at[pl.ds(j * IR, IR)], buf[3]
            ).wait()

    def wait_store(buf):
        pltpu.make_async_copy(buf[1], out_hbm.at[pl.ds(0, CH)], buf[4]).wait()

    issue_idx(0, bufs[0])
    issue_idx(1, bufs[1])
    wait_idx(bufs[0])
    fire_gathers(bufs[0])

    def body(kk, carry):
        for b in (0, 1):
            k = 2 * kk + b
            bx = bufs[b]
            by = bufs[1 - b]

            @pl.when(k < n)
            def _():
                wait_gathers(bx)

                @pl.when(k + 2 < n)
                def _():
                    issue_idx(k + 2, bx)

                @pl.when(k + 1 < n)
                def _():
                    wait_idx(by)

                    @pl.when(k >= 1)
                    def _():
                        wait_store(by)

                    fire_gathers(by)

                base = pl.multiple_of(ch_of(k) * CH, 8)
                pltpu.async_copy(bx[1], out_hbm.at[pl.ds(base, CH)], bx[4])

        return carry

    lax.fori_loop(0, (n + 1) // 2, body, 0)
    wait_store(bufs[0])
    wait_store(bufs[1])


BN = 1000


def _prep_body(x, w1a, b1, y):
    y[...] = jnp.dot(x[...], w1a[...], preferred_element_type=jnp.float32) + b1[...]


def _prep(x, w1a, b1):
    grid = (N // BN,)
    return pl.pallas_call(
        _prep_body,
        grid=grid,
        in_specs=[
            pl.BlockSpec((BN, D), lambda i: (i, 0)),
            pl.BlockSpec((D, MSG), lambda i: (0, 0)),
            pl.BlockSpec((1, MSG), lambda i: (0, 0)),
        ],
        out_specs=pl.BlockSpec((BN, MSG), lambda i: (i, 0)),
        out_shape=jax.ShapeDtypeStruct((N, MSG), jnp.float32),
    )(x, w1a, b1)


BE = 2000


def _mlp_body(yg, ea, w1b, w2, b2, m):
    h = yg[...] + jnp.dot(ea[...], w1b[...], preferred_element_type=jnp.float32)
    h = jnp.maximum(h, 0.0)
    m[...] = jnp.dot(h, w2[...], preferred_element_type=jnp.float32) + b2[...]


def _mlp(yg, ea, w1b, w2, b2):
    grid = (E2 // BE,)
    return pl.pallas_call(
        _mlp_body,
        grid=grid,
        in_specs=[
            pl.BlockSpec((BE, MSG), lambda i: (i, 0)),
            pl.BlockSpec((BE, D), lambda i: (i, 0)),
            pl.BlockSpec((D, MSG), lambda i: (0, 0)),
            pl.BlockSpec((MSG, D), lambda i: (0, 0)),
            pl.BlockSpec((1, D), lambda i: (0, 0)),
        ],
        out_specs=pl.BlockSpec((BE, D), lambda i: (i, 0)),
        out_shape=jax.ShapeDtypeStruct((E2, D), jnp.float32),
    )(yg, ea, w1b, w2, b2)


@functools.partial(
    pl.kernel,
    out_type=jax.ShapeDtypeStruct((N, D), jnp.float32),
    mesh=_mesh,
    scratch_types=[
        pltpu.VMEM_SHARED((NACC, D), jnp.float32),
        pltpu.VMEM((KC, IR), jnp.int32),
        pltpu.VMEM((KC, IR), jnp.int32),
        pltpu.VMEM((CH, D), jnp.float32),
        pltpu.VMEM((CH, D), jnp.float32),
        pltpu.VMEM((IR,), jnp.int32),
        pltpu.VMEM((ZB, D), jnp.float32),
        pltpu.SemaphoreType.DMA,
        pltpu.SemaphoreType.DMA,
        pltpu.SemaphoreType.DMA,
        pltpu.SemaphoreType.DMA,
    ],
)
def _scatter(
    m_hbm, dst_hbm, out_hbm,
    agg_sp, idx_a, idx_b, rows_a, rows_b, idx1d, zb_v,
    semi_a, semi_b, semr_a, semr_b,
):
    c = lax.axis_index("c")
    s = lax.axis_index("s")
    node0 = c * NLOC

    bufs = (
        (idx_a, rows_a, semi_a, semr_a),
        (idx_b, rows_b, semi_b, semr_b),
    )
    n = (NCHH - 1 - s) // NS + 1

    def ch_of(k):
        return s + k * NS

    def issue(k, buf):
        ch = ch_of(k)
        base = pl.multiple_of(ch * CH, 8)
        pltpu.async_copy(dst_hbm.at[ch], buf[0], buf[2])
        pltpu.async_copy(m_hbm.at[pl.ds(base, CH)], buf[1], buf[3])

    def wait(buf):
        pltpu.make_async_copy(dst_hbm.at[0], buf[0], buf[2]).wait()
        pltpu.make_async_copy(m_hbm.at[pl.ds(0, CH)], buf[1], buf[3]).wait()

    issue(0, bufs[0])
    issue(1, bufs[1])

    def z_fill(i, carry):
        zb_v[i // 8, pl.ds((i % 8) * 16, 16)] = jnp.zeros((16,), jnp.float32)
        return carry

    lax.fori_loop(0, ZB * (D // 16), z_fill, 0)
    row0 = pl.multiple_of(s * RPT, 8)

    def z_copy(i, carry):
        off = pl.multiple_of(row0 + i * ZB, 8)
        pltpu.sync_copy(zb_v, agg_sp.at[pl.ds(off, ZB)])
        return carry

    lax.fori_loop(0, RPT // ZB, z_copy, 0)

    @pl.when(s == NS - 1)
    def _():
        pltpu.sync_copy(
            zb_v.at[pl.ds(0, TAIL_ZREM)],
            agg_sp.at[pl.ds(TAIL_OFF + 3 * ZB, TAIL_ZREM)],
        )

    plsc.subcore_barrier()

    def process(buf):
        wait(buf)
        for j in range(KC):
            for g in range(IR // 16):
                v = buf[0][j, pl.ds(g * 16, 16)] - node0
                ok = (v >= 0) & (v < NLOC)
                idx1d[pl.ds(g * 16, 16)] = jnp.where(ok, v, DUMMY)
            pltpu.sync_copy(
                buf[1].at[pl.ds(j * IR, IR)], agg_sp.at[idx1d], add=True
            )

    def body(kk, carry):
        for b in (0, 1):
            k = 2 * kk + b

            @pl.when(k < n)
            def _():
                process(bufs[b])

                @pl.when(k + 2 < n)
                def _():
                    issue(k + 2, bufs[b])

        return carry

    lax.fori_loop(0, (n + 1) // 2, body, 0)
    plsc.subcore_barrier()

    out0 = pl.multiple_of(node0 + row0, 8)

    @pl.when(s < NS - 1)
    def _():
        pltpu.sync_copy(
            agg_sp.at[pl.ds(row0, RPT)], out_hbm.at[pl.ds(out0, RPT)]
        )

    @pl.when(s == NS - 1)
    def _():
        pltpu.sync_copy(
            agg_sp.at[pl.ds(TAIL_OFF, TAIL_W)],
            out_hbm.at[pl.ds(pl.multiple_of(node0 + TAIL_OFF, 8), TAIL_W)],
        )


def _final_body(agg_a, agg_b, x, ws, bs, out):
    t = agg_a[...] + agg_b[...]
    t = t + jnp.dot(x[...], ws[...], preferred_element_type=jnp.float32)
    out[...] = jnp.maximum(t + bs[...], 0.0)


def _final(agg_a, agg_b, x, ws, bs):
    grid = (N // BN,)
    return pl.pallas_call(
        _final_body,
        grid=grid,
        in_specs=[
            pl.BlockSpec((BN, D), lambda i: (i, 0)),
            pl.BlockSpec((BN, D), lambda i: (i, 0)),
            pl.BlockSpec((BN, D), lambda i: (i, 0)),
            pl.BlockSpec((D, D), lambda i: (0, 0)),
            pl.BlockSpec((1, D), lambda i: (0, 0)),
        ],
        out_specs=pl.BlockSpec((BN, D), lambda i: (i, 0)),
        out_shape=jax.ShapeDtypeStruct((N, D), jnp.float32),
    )(agg_a, agg_b, x, ws, bs)


def kernel(x, edge_index, edge_attr, W1, b1, W2, b2, Ws, bs):
    src = edge_index[0].astype(jnp.int32).reshape(NH, NCHH, KC, IR)
    dst = edge_index[1].astype(jnp.int32).reshape(NH, NCHH, KC, IR)
    w1a = W1[:D]
    w1b = W1[D:]
    b1r = b1.reshape(1, MSG)
    b2r = b2.reshape(1, D)
    bsr = bs.reshape(1, D)
    ea = edge_attr.reshape(NH, E2, D)

    y = _prep(x, w1a, b1r)
    yg0 = _gather(y, src[0])
    yg1 = _gather(y, src[1])
    m0 = _mlp(yg0, ea[0], w1b, W2, b2r)
    m1 = _mlp(yg1, ea[1], w1b, W2, b2r)
    agg0 = _scatter(m0, dst[0])
    agg1 = _scatter(m1, dst[1])
    x_up = _final(agg0, agg1, x, Ws, bsr)
    return (x_up, edge_attr)


# speedup vs baseline: 1.0062x; 1.0062x over previous
import functools

import jax
import jax.numpy as jnp
from jax import lax
from jax.experimental import pallas as pl
from jax.experimental.pallas import tpu as pltpu
from jax.experimental.pallas import tpu_sc as plsc

N = 10000
E = 320000
D = 128
MSG = 128

NC = 2
NS = 16
NW = NC * NS

IR = 128
KC = 2
CH = KC * IR
NCHP = E // CH

NLOC = N // NC
NACC = NLOC + 8
DUMMY = NLOC
RPT = 312
TAIL_OFF = (NS - 1) * RPT
TAIL_W = NLOC - TAIL_OFF
ZB = 104
TAIL_ZREM = NACC - TAIL_OFF - 3 * ZB

_mesh = plsc.VectorSubcoreMesh(core_axis_name="c", subcore_axis_name="s")


@functools.partial(
    pl.kernel,
    out_type=jax.ShapeDtypeStruct((E, D), jnp.float32),
    mesh=_mesh,
    scratch_types=[
        pltpu.VMEM((KC, IR), jnp.int32),
        pltpu.VMEM((KC, IR), jnp.int32),
        pltpu.VMEM((CH, D), jnp.float32),
        pltpu.VMEM((CH, D), jnp.float32),
        pltpu.SemaphoreType.DMA,
        pltpu.SemaphoreType.DMA,
        pltpu.SemaphoreType.DMA,
        pltpu.SemaphoreType.DMA,
        pltpu.SemaphoreType.DMA,
        pltpu.SemaphoreType.DMA,
    ],
)
def _gather(
    x_hbm, src_hbm, out_hbm,
    idx_a, idx_b, rows_a, rows_b,
    semi_a, semi_b, semg_a, semg_b, semo_a, semo_b,
):
    c = lax.axis_index("c")
    s = lax.axis_index("s")
    wid = s * NC + c
    n = (NCHP - 1 - wid) // NW + 1

    bufs = (
        (idx_a, rows_a, semi_a, semg_a, semo_a),
        (idx_b, rows_b, semi_b, semg_b, semo_b),
    )

    def ch_of(k):
        return wid + k * NW

    def issue_idx(k, buf):
        pltpu.async_copy(src_hbm.at[ch_of(k)], buf[0], buf[2])

    def fire_gathers(buf):
        for j in range(KC):
            pltpu.async_copy(
                x_hbm.at[buf[0].at[j]], buf[1].at[pl.ds(j * IR, IR)], buf[3]
            )

    def wait_idx(buf):
        pltpu.make_async_copy(src_hbm.at[0], buf[0], buf[2]).wait()

    def wait_gathers(buf):
        for j in range(KC):
            pltpu.make_async_copy(
                x_hbm.at[pl.ds(0, IR)], buf[1].at[pl.ds(j * IR, IR)], buf[3]
            ).wait()

    def wait_store(buf):
        pltpu.make_async_copy(buf[1], out_hbm.at[pl.ds(0, CH)], buf[4]).wait()

    issue_idx(0, bufs[0])
    issue_idx(1, bufs[1])
    wait_idx(bufs[0])
    fire_gathers(bufs[0])

    def body(kk, carry):
        for b in (0, 1):
            k = 2 * kk + b
            bx = bufs[b]
            by = bufs[1 - b]

            @pl.when(k < n)
            def _():
                wait_gathers(bx)

                @pl.when(k + 2 < n)
                def _():
                    issue_idx(k + 2, bx)

                @pl.when(k + 1 < n)
                def _():
                    wait_idx(by)

                    @pl.when(k >= 1)
                    def _():
                        wait_store(by)

                    fire_gathers(by)

                base = pl.multiple_of(ch_of(k) * CH, 8)
                pltpu.async_copy(bx[1], out_hbm.at[pl.ds(base, CH)], bx[4])

        return carry

    lax.fori_loop(0, (n + 1) // 2, body, 0)
    wait_store(bufs[0])
    wait_store(bufs[1])


BN = 1000


def _prep_body(x, w1a, b1, y):
    y[...] = jnp.dot(x[...], w1a[...], preferred_element_type=jnp.float32) + b1[...]


def _prep(x, w1a, b1):
    grid = (N // BN,)
    return pl.pallas_call(
        _prep_body,
        grid=grid,
        in_specs=[
            pl.BlockSpec((BN, D), lambda i: (i, 0)),
            pl.BlockSpec((D, MSG), lambda i: (0, 0)),
            pl.BlockSpec((1, MSG), lambda i: (0, 0)),
        ],
        out_specs=pl.BlockSpec((BN, MSG), lambda i: (i, 0)),
        out_shape=jax.ShapeDtypeStruct((N, MSG), jnp.float32),
    )(x, w1a, b1)


BE = 2000


def _mlp_body(yg, ea, w1b, w2, b2, m):
    h = yg[...] + jnp.dot(ea[...], w1b[...], preferred_element_type=jnp.float32)
    h = jnp.maximum(h, 0.0)
    m[...] = jnp.dot(h, w2[...], preferred_element_type=jnp.float32) + b2[...]


def _mlp(yg, ea, w1b, w2, b2):
    grid = (E // BE,)
    return pl.pallas_call(
        _mlp_body,
        grid=grid,
        in_specs=[
            pl.BlockSpec((BE, MSG), lambda i: (i, 0)),
            pl.BlockSpec((BE, D), lambda i: (i, 0)),
            pl.BlockSpec((D, MSG), lambda i: (0, 0)),
            pl.BlockSpec((MSG, D), lambda i: (0, 0)),
            pl.BlockSpec((1, D), lambda i: (0, 0)),
        ],
        out_specs=pl.BlockSpec((BE, D), lambda i: (i, 0)),
        out_shape=jax.ShapeDtypeStruct((E, D), jnp.float32),
    )(yg, ea, w1b, w2, b2)


@functools.partial(
    pl.kernel,
    out_type=jax.ShapeDtypeStruct((N, D), jnp.float32),
    mesh=_mesh,
    scratch_types=[
        pltpu.VMEM_SHARED((NACC, D), jnp.float32),
        pltpu.VMEM((KC, IR), jnp.int32),
        pltpu.VMEM((KC, IR), jnp.int32),
        pltpu.VMEM((CH, D), jnp.float32),
        pltpu.VMEM((CH, D), jnp.float32),
        pltpu.VMEM((IR,), jnp.int32),
        pltpu.VMEM((ZB, D), jnp.float32),
        pltpu.SemaphoreType.DMA,
        pltpu.SemaphoreType.DMA,
        pltpu.SemaphoreType.DMA,
        pltpu.SemaphoreType.DMA,
    ],
)
def _scatter(
    m_hbm, dst_hbm, out_hbm,
    agg_sp, idx_a, idx_b, rows_a, rows_b, idx1d, zb_v,
    semi_a, semi_b, semr_a, semr_b,
):
    c = lax.axis_index("c")
    s = lax.axis_index("s")
    node0 = c * NLOC

    bufs = (
        (idx_a, rows_a, semi_a, semr_a),
        (idx_b, rows_b, semi_b, semr_b),
    )
    n = (NCHP - 1 - s) // NS + 1

    def ch_of(k):
        return s + k * NS

    def issue(k, buf):
        ch = ch_of(k)
        base = pl.multiple_of(ch * CH, 8)
        pltpu.async_copy(dst_hbm.at[ch], buf[0], buf[2])
        pltpu.async_copy(m_hbm.at[pl.ds(base, CH)], buf[1], buf[3])

    def wait(buf):
        pltpu.make_async_copy(dst_hbm.at[0], buf[0], buf[2]).wait()
        pltpu.make_async_copy(m_hbm.at[pl.ds(0, CH)], buf[1], buf[3]).wait()

    issue(0, bufs[0])
    issue(1, bufs[1])

    def z_fill(i, carry):
        zb_v[i // 8, pl.ds((i % 8) * 16, 16)] = jnp.zeros((16,), jnp.float32)
        return carry

    lax.fori_loop(0, ZB * (D // 16), z_fill, 0)
    row0 = pl.multiple_of(s * RPT, 8)

    def z_copy(i, carry):
        off = pl.multiple_of(row0 + i * ZB, 8)
        pltpu.sync_copy(zb_v, agg_sp.at[pl.ds(off, ZB)])
        return carry

    lax.fori_loop(0, RPT // ZB, z_copy, 0)

    @pl.when(s == NS - 1)
    def _():
        pltpu.sync_copy(
            zb_v.at[pl.ds(0, TAIL_ZREM)],
            agg_sp.at[pl.ds(TAIL_OFF + 3 * ZB, TAIL_ZREM)],
        )

    plsc.subcore_barrier()

    def process(buf):
        wait(buf)
        for j in range(KC):
            for g in range(IR // 16):
                v = buf[0][j, pl.ds(g * 16, 16)] - node0
                ok = (v >= 0) & (v < NLOC)
                idx1d[pl.ds(g * 16, 16)] = jnp.where(ok, v, DUMMY)
            pltpu.sync_copy(
                buf[1].at[pl.ds(j * IR, IR)], agg_sp.at[idx1d], add=True
            )

    def body(kk, carry):
        for b in (0, 1):
            k = 2 * kk + b

            @pl.when(k < n)
            def _():
                process(bufs[b])

                @pl.when(k + 2 < n)
                def _():
                    issue(k + 2, bufs[b])

        return carry

    lax.fori_loop(0, (n + 1) // 2, body, 0)
    plsc.subcore_barrier()

    out0 = pl.multiple_of(node0 + row0, 8)

    @pl.when(s < NS - 1)
    def _():
        pltpu.sync_copy(
            agg_sp.at[pl.ds(row0, RPT)], out_hbm.at[pl.ds(out0, RPT)]
        )

    @pl.when(s == NS - 1)
    def _():
        pltpu.sync_copy(
            agg_sp.at[pl.ds(TAIL_OFF, TAIL_W)],
            out_hbm.at[pl.ds(pl.multiple_of(node0 + TAIL_OFF, 8), TAIL_W)],
        )


def _final_body(agg, x, ws, bs, out):
    t = agg[...] + jnp.dot(x[...], ws[...], preferred_element_type=jnp.float32)
    out[...] = jnp.maximum(t + bs[...], 0.0)


def _final(agg, x, ws, bs):
    grid = (N // BN,)
    return pl.pallas_call(
        _final_body,
        grid=grid,
        in_specs=[
            pl.BlockSpec((BN, D), lambda i: (i, 0)),
            pl.BlockSpec((BN, D), lambda i: (i, 0)),
            pl.BlockSpec((D, D), lambda i: (0, 0)),
            pl.BlockSpec((1, D), lambda i: (0, 0)),
        ],
        out_specs=pl.BlockSpec((BN, D), lambda i: (i, 0)),
        out_shape=jax.ShapeDtypeStruct((N, D), jnp.float32),
    )(agg, x, ws, bs)


def kernel(x, edge_index, edge_attr, W1, b1, W2, b2, Ws, bs):
    src = edge_index[0].astype(jnp.int32).reshape(NCHP, KC, IR)
    dst = edge_index[1].astype(jnp.int32).reshape(NCHP, KC, IR)
    w1a = W1[:D]
    w1b = W1[D:]
    b1r = b1.reshape(1, MSG)
    b2r = b2.reshape(1, D)
    bsr = bs.reshape(1, D)

    y = _prep(x, w1a, b1r)
    yg = _gather(y, src)
    m = _mlp(yg, edge_attr, w1b, W2, b2r)
    agg = _scatter(m, dst)
    x_up = _final(agg, x, Ws, bsr)
    return (x_up, edge_attr)
